# baseline (speedup 1.0000x reference)
import functools

import jax
import jax.numpy as jnp
from jax import lax
from jax.experimental import pallas as pl
from jax.experimental.pallas import tpu as pltpu
from jax.experimental.pallas import tpu_sc as plsc

VOCAB = 100000
DIM = 300
DIMP = 304
B = 1024
L = 200

NC = 2
NS = 16
NW = NC * NS

N = B * L
N_PER_W = N // NW
CHUNK = 128
N_CHUNKS = N_PER_W // CHUNK


def _make_sc_gather():
  mesh = plsc.VectorSubcoreMesh(core_axis_name="c", subcore_axis_name="s")

  @functools.partial(
      pl.kernel,
      mesh=mesh,
      compiler_params=pltpu.CompilerParams(use_tc_tiling_on_sc=False),
      out_type=jax.ShapeDtypeStruct((N, DIMP), jnp.float32),
      scratch_types=[
          pltpu.VMEM((N_CHUNKS, CHUNK), jnp.int32),
          pltpu.VMEM((CHUNK, DIMP), jnp.float32),
          pltpu.SemaphoreType.DMA,
      ],
  )
  def sc_gather(table_hbm, idx_hbm, out_hbm, idx_v, rows_v, sem):
    wid = lax.axis_index("s") * NC + lax.axis_index("c")
    base = wid * N_PER_W
    pltpu.sync_copy(idx_hbm.at[wid], idx_v)

    def body(c, carry):
      pltpu.async_copy(table_hbm.at[idx_v.at[c]], rows_v, sem).wait()
      pltpu.sync_copy(rows_v, out_hbm.at[pl.ds(base + c * CHUNK, CHUNK)])
      return carry

    lax.fori_loop(0, N_CHUNKS, body, 0)

  return sc_gather


_sc_gather = _make_sc_gather()


def kernel(embedding_table, word_sequences):
  table_p = jnp.pad(embedding_table, ((0, 0), (0, DIMP - DIM)))
  idx = word_sequences.reshape(NW, N_CHUNKS, CHUNK)
  out = _sc_gather(table_p, idx)
  return out[:, :DIM].reshape(B, L, DIM)


# speedup vs baseline: 1.4501x; 1.4501x over previous
import functools

import jax
import jax.numpy as jnp
from jax import lax
from jax.experimental import pallas as pl
from jax.experimental.pallas import tpu as pltpu
from jax.experimental.pallas import tpu_sc as plsc

VOCAB = 100000
DIM = 300
DIMP = 384
B = 1024
L = 200

NC = 2
NS = 16
NW = NC * NS

N = B * L
N_PER_W = N // NW
CHUNK = 128
N_CHUNKS = N_PER_W // CHUNK


def _make_sc_gather():
  mesh = plsc.VectorSubcoreMesh(core_axis_name="c", subcore_axis_name="s")

  @functools.partial(
      pl.kernel,
      mesh=mesh,
      compiler_params=pltpu.CompilerParams(use_tc_tiling_on_sc=True),
      out_type=jax.ShapeDtypeStruct((N, DIMP), jnp.float32),
      scratch_types=[
          pltpu.VMEM((N_CHUNKS, CHUNK), jnp.int32),
          pltpu.VMEM((CHUNK, DIMP), jnp.float32),
          pltpu.SemaphoreType.DMA,
      ],
  )
  def sc_gather(table_hbm, idx_hbm, out_hbm, idx_v, rows_v, sem):
    wid = lax.axis_index("s") * NC + lax.axis_index("c")
    base = wid * N_PER_W
    pltpu.sync_copy(idx_hbm.at[wid], idx_v)

    def body(c, carry):
      pltpu.async_copy(table_hbm.at[idx_v.at[c]], rows_v, sem).wait()
      pltpu.sync_copy(rows_v, out_hbm.at[pl.ds(base + c * CHUNK, CHUNK)])
      return carry

    lax.fori_loop(0, N_CHUNKS, body, 0)

  return sc_gather


_sc_gather = _make_sc_gather()


def kernel(embedding_table, word_sequences):
  table_p = jnp.pad(embedding_table, ((0, 0), (0, DIMP - DIM)))
  idx = word_sequences.reshape(NW, N_CHUNKS, CHUNK)
  out = _sc_gather(table_p, idx)
  return out[:, :DIM].reshape(B, L, DIM)


# speedup vs baseline: 2.1068x; 1.4529x over previous
import functools

import jax
import jax.numpy as jnp
from jax import lax
from jax.experimental import pallas as pl
from jax.experimental.pallas import tpu as pltpu
from jax.experimental.pallas import tpu_sc as plsc

VOCAB = 100000
DIM = 300
DM = 256
DT = 128
B = 1024
L = 200

NC = 2
NS = 16
NW = NC * NS

N = B * L
N_PER_W = N // NW
CHUNK = 128
N_CHUNKS = N_PER_W // CHUNK


def _make_sc_gather():
  mesh = plsc.VectorSubcoreMesh(core_axis_name="c", subcore_axis_name="s")

  @functools.partial(
      pl.kernel,
      mesh=mesh,
      compiler_params=pltpu.CompilerParams(use_tc_tiling_on_sc=True),
      out_type=(jax.ShapeDtypeStruct((N, DIM), jnp.float32),
                jax.ShapeDtypeStruct((N, DT), jnp.float32)),
      scratch_types=[
          pltpu.VMEM((N_CHUNKS, CHUNK), jnp.int32),
          pltpu.VMEM((CHUNK, DM), jnp.float32),
          pltpu.VMEM((CHUNK, DT), jnp.float32),
          pltpu.SemaphoreType.DMA,
          pltpu.SemaphoreType.DMA,
      ],
  )
  def sc_gather(table_hbm, tail_hbm, idx_hbm, out_hbm, outt_hbm,
                idx_v, main_v, tail_v, sem_a, sem_b):
    wid = lax.axis_index("s") * NC + lax.axis_index("c")
    base = wid * N_PER_W
    pltpu.sync_copy(idx_hbm.at[wid], idx_v)

    def body(c, carry):
      rb = base + c * CHUNK
      a = pltpu.async_copy(
          table_hbm.at[idx_v.at[c], pl.ds(0, DM)], main_v, sem_a)
      b = pltpu.async_copy(tail_hbm.at[idx_v.at[c]], tail_v, sem_b)
      a.wait()
      pltpu.sync_copy(main_v, out_hbm.at[pl.ds(rb, CHUNK), pl.ds(0, DM)])
      b.wait()
      pltpu.sync_copy(tail_v, outt_hbm.at[pl.ds(rb, CHUNK)])
      return carry

    lax.fori_loop(0, N_CHUNKS, body, 0)

  return sc_gather


_sc_gather = _make_sc_gather()


def kernel(embedding_table, word_sequences):
  tail_p = jnp.pad(embedding_table[:, DM:], ((0, 0), (0, DT - (DIM - DM))))
  idx = word_sequences.reshape(NW, N_CHUNKS, CHUNK)
  out, outt = _sc_gather(embedding_table, tail_p, idx)
  out = lax.dynamic_update_slice(out, outt[:, :DIM - DM], (0, DM))
  return out.reshape(B, L, DIM)
